# Initial kernel scaffold; baseline (speedup 1.0000x reference)
import jax
import jax.numpy as jnp
from jax.experimental import pallas as pl


def kernel(embedding_weight, edge_index_mp):
    raise NotImplementedError("write your pallas kernel here")


# speedup vs baseline: 8.2572x; 8.2572x over previous
import functools

import jax
import jax.numpy as jnp
from jax import lax
from jax.experimental import pallas as pl
from jax.experimental.pallas import tpu as pltpu
from jax.experimental.pallas import tpu_sc as plsc

N = 10000
D = 128
E = 320000
NUM_LAYERS = 3

NC = 2
NS = 16
NW = NC * NS
EPT = E // NW
K = 80
NCHUNK = EPT // K
RPT = N // NS
ZROWS = 125

_mesh = plsc.VectorSubcoreMesh(
    core_axis_name="c", subcore_axis_name="s", num_cores=NC, num_subcores=NS)

_sc_params = pltpu.CompilerParams(use_tc_tiling_on_sc=False)


def _tile_base(c, s):
    return (c * NS + s) * EPT


@functools.partial(
    pl.kernel,
    out_type=jax.ShapeDtypeStruct((NC, N, 16), jnp.float32),
    mesh=_mesh,
    scratch_types=[
        pltpu.VMEM_SHARED((N, 16), jnp.float32),
        pltpu.VMEM((K,), jnp.int32),
        pltpu.VMEM((K, 16), jnp.float32),
        pltpu.VMEM((RPT, 16), jnp.float32),
    ],
    compiler_params=_sc_params,
)
def _deg_sc(col_hbm, deg_hbm, acc, cidx, ones, zbuf):
    c = lax.axis_index("c")
    s = lax.axis_index("s")

    ones16 = jnp.full((16,), 1.0, jnp.float32)
    zeros16 = jnp.zeros((16,), jnp.float32)

    def fill(i, _):
        ones[i, :] = ones16
        return 0
    lax.fori_loop(0, K, fill, 0)

    def zfill(i, _):
        zbuf[i, :] = zeros16
        return 0
    lax.fori_loop(0, RPT, zfill, 0)

    pltpu.sync_copy(zbuf, acc.at[pl.ds(s * RPT, RPT)])
    plsc.subcore_barrier()

    ebase = _tile_base(c, s)

    def chunk(j, _):
        pltpu.sync_copy(col_hbm.at[pl.ds(ebase + j * K, K)], cidx)
        pltpu.sync_copy(ones, acc.at[cidx], add=True)
        return 0
    lax.fori_loop(0, NCHUNK, chunk, 0)

    plsc.subcore_barrier()
    pltpu.sync_copy(acc.at[pl.ds(s * RPT, RPT)],
                    deg_hbm.at[c, pl.ds(s * RPT, RPT)])


@functools.partial(
    pl.kernel,
    out_type=jax.ShapeDtypeStruct((NC, N, D), jnp.float32),
    mesh=_mesh,
    scratch_types=[
        pltpu.VMEM_SHARED((N, D), jnp.float32),
        pltpu.VMEM((K,), jnp.int32),
        pltpu.VMEM((K,), jnp.int32),
        pltpu.VMEM((K, D), jnp.float32),
        pltpu.VMEM((ZROWS, D), jnp.float32),
        pltpu.SemaphoreType.DMA,
    ],
    compiler_params=_sc_params,
)
def _layer_sc(s_hbm, row_hbm, col_hbm, q_hbm, acc, ridx, cidx, buf, zbuf, sem):
    c = lax.axis_index("c")
    s = lax.axis_index("s")

    zeros16 = jnp.zeros((16,), jnp.float32)

    def zfill(i, _):
        for d in range(D // 16):
            zbuf[i, pl.ds(d * 16, 16)] = zeros16
        return 0
    lax.fori_loop(0, ZROWS, zfill, 0)

    for j in range(RPT // ZROWS):
        pltpu.sync_copy(zbuf, acc.at[pl.ds(s * RPT + j * ZROWS, ZROWS)])
    plsc.subcore_barrier()

    ebase = _tile_base(c, s)

    def chunk(j, _):
        off = ebase + j * K
        pltpu.sync_copy(row_hbm.at[pl.ds(off, K)], ridx)
        pltpu.sync_copy(col_hbm.at[pl.ds(off, K)], cidx)
        pltpu.async_copy(s_hbm.at[ridx], buf, sem).wait()
        pltpu.sync_copy(buf, acc.at[cidx], add=True)
        return 0
    lax.fori_loop(0, NCHUNK, chunk, 0)

    plsc.subcore_barrier()
    for j in range(RPT // ZROWS):
        start = s * RPT + j * ZROWS
        pltpu.sync_copy(acc.at[pl.ds(start, ZROWS)],
                        q_hbm.at[c, pl.ds(start, ZROWS)])


_BLK = 2000


def _prep_body(p0_ref, p1_ref, x_ref, disb_ref, s_ref):
    deg = p0_ref[:, 0:1] + p1_ref[:, 0:1]
    dis = jnp.where(deg > 0.0, 1.0 / jnp.sqrt(deg), 0.0)
    dis_b = jnp.broadcast_to(dis, (_BLK, D))
    disb_ref[...] = dis_b
    s_ref[...] = dis_b * x_ref[...]


def _prep_tc(p0, p1, x0):
    return pl.pallas_call(
        _prep_body,
        grid=(N // _BLK,),
        in_specs=[
            pl.BlockSpec((_BLK, 16), lambda i: (i, 0)),
            pl.BlockSpec((_BLK, 16), lambda i: (i, 0)),
            pl.BlockSpec((_BLK, D), lambda i: (i, 0)),
        ],
        out_specs=[
            pl.BlockSpec((_BLK, D), lambda i: (i, 0)),
            pl.BlockSpec((_BLK, D), lambda i: (i, 0)),
        ],
        out_shape=[
            jax.ShapeDtypeStruct((N, D), jnp.float32),
            jax.ShapeDtypeStruct((N, D), jnp.float32),
        ],
    )(p0, p1, x0)


def _combine_body(q0_ref, q1_ref, disb_ref, acc_ref, accn_ref, sn_ref):
    dis_b = disb_ref[...]
    x_next = dis_b * (q0_ref[...] + q1_ref[...])
    accn_ref[...] = acc_ref[...] + x_next
    sn_ref[...] = dis_b * x_next


def _combine_tc(q0, q1, dis_b, acc):
    blk = pl.BlockSpec((_BLK, D), lambda i: (i, 0))
    return pl.pallas_call(
        _combine_body,
        grid=(N // _BLK,),
        in_specs=[blk, blk, blk, blk],
        out_specs=[blk, blk],
        out_shape=[
            jax.ShapeDtypeStruct((N, D), jnp.float32),
            jax.ShapeDtypeStruct((N, D), jnp.float32),
        ],
    )(q0, q1, dis_b, acc)


def _final_body(q0_ref, q1_ref, disb_ref, acc_ref, out_ref):
    x_next = disb_ref[...] * (q0_ref[...] + q1_ref[...])
    out_ref[...] = (acc_ref[...] + x_next) * (1.0 / (NUM_LAYERS + 1))


def _final_tc(q0, q1, dis_b, acc):
    blk = pl.BlockSpec((_BLK, D), lambda i: (i, 0))
    return pl.pallas_call(
        _final_body,
        grid=(N // _BLK,),
        in_specs=[blk, blk, blk, blk],
        out_specs=blk,
        out_shape=jax.ShapeDtypeStruct((N, D), jnp.float32),
    )(q0, q1, dis_b, acc)


def kernel(embedding_weight, edge_index_mp):
    x0 = embedding_weight
    row = edge_index_mp[0]
    col = edge_index_mp[1]

    degq = _deg_sc(col)
    dis_b, s = _prep_tc(degq[0], degq[1], x0)
    acc = x0
    for l in range(NUM_LAYERS):
        q = _layer_sc(s, row, col)
        if l < NUM_LAYERS - 1:
            acc, s = _combine_tc(q[0], q[1], dis_b, acc)
        else:
            out = _final_tc(q[0], q[1], dis_b, acc)
    return out


# speedup vs baseline: 15.8055x; 1.9142x over previous
import functools

import jax
import jax.numpy as jnp
from jax import lax
from jax.experimental import pallas as pl
from jax.experimental.pallas import tpu as pltpu
from jax.experimental.pallas import tpu_sc as plsc

N = 10000
D = 128
E = 320000
NUM_LAYERS = 3

NC = 2
NS = 16
NW = NC * NS
EPT = E // NW
K = 100
NCHUNK = EPT // K
NBUF = 2
RPT = N // NS

_mesh = plsc.VectorSubcoreMesh(
    core_axis_name="c", subcore_axis_name="s", num_cores=NC, num_subcores=NS)

_sc_params = pltpu.CompilerParams(use_tc_tiling_on_sc=False)


@functools.partial(
    pl.kernel,
    out_type=jax.ShapeDtypeStruct((NC, N, 16), jnp.float32),
    mesh=_mesh,
    scratch_types=[
        pltpu.VMEM_SHARED((N, 16), jnp.float32),
        pltpu.VMEM((NCHUNK, K), jnp.int32),
        pltpu.VMEM((K, 16), jnp.float32),
        pltpu.SemaphoreType.DMA,
    ],
    compiler_params=_sc_params,
)
def _deg_sc(col_hbm, z16_hbm, deg_hbm, acc, cidx, ones, sem):
    c = lax.axis_index("c")
    s = lax.axis_index("s")
    tid = c * NS + s

    pltpu.sync_copy(col_hbm.at[tid], cidx)

    ones16 = jnp.full((16,), 1.0, jnp.float32)

    def fill(i, _):
        ones[i, :] = ones16
        return 0
    lax.fori_loop(0, K, fill, 0)

    pltpu.sync_copy(z16_hbm, acc.at[pl.ds(s * RPT, RPT)])
    plsc.subcore_barrier()

    def chunk(j, _):
        pltpu.async_copy(ones, acc.at[cidx.at[j]], sem, add=True)
        return 0
    lax.fori_loop(0, NCHUNK, chunk, 0)

    def drain(j, _):
        pltpu.make_async_copy(ones, acc.at[cidx.at[j]], sem).wait()
        return 0
    lax.fori_loop(0, NCHUNK, drain, 0)

    plsc.subcore_barrier()
    pltpu.sync_copy(acc.at[pl.ds(s * RPT, RPT)],
                    deg_hbm.at[c, pl.ds(s * RPT, RPT)])


@functools.partial(
    pl.kernel,
    out_type=jax.ShapeDtypeStruct((NC, N, D), jnp.float32),
    mesh=_mesh,
    scratch_types=[
        pltpu.VMEM_SHARED((N, D), jnp.float32),
        pltpu.VMEM((NCHUNK, K), jnp.int32),
        pltpu.VMEM((NCHUNK, K), jnp.int32),
        [pltpu.VMEM((K, D), jnp.float32) for _ in range(NBUF)],
        [pltpu.SemaphoreType.DMA for _ in range(NBUF)],
        [pltpu.SemaphoreType.DMA for _ in range(NBUF)],
    ],
    compiler_params=_sc_params,
)
def _layer_sc(s_hbm, row_hbm, col_hbm, z_hbm, q_hbm, acc, ridx, cidx, bufs,
              gsems, ssems):
    c = lax.axis_index("c")
    s = lax.axis_index("s")
    tid = c * NS + s

    pltpu.sync_copy(row_hbm.at[tid], ridx)
    pltpu.sync_copy(col_hbm.at[tid], cidx)

    def gather_wait(j, b):
        pltpu.make_async_copy(s_hbm.at[ridx.at[j]], bufs[b], gsems[b]).wait()

    def scatter_wait(j, b):
        pltpu.make_async_copy(bufs[b], acc.at[cidx.at[j]], ssems[b]).wait()

    for b in range(NBUF):
        pltpu.async_copy(s_hbm.at[ridx.at[b]], bufs[b], gsems[b])

    pltpu.sync_copy(z_hbm, acc.at[pl.ds(s * RPT, RPT)])
    plsc.subcore_barrier()

    def body(t, _):
        for b in range(NBUF):
            j = t * NBUF + b
            gather_wait(j, b)
            pltpu.async_copy(bufs[b], acc.at[cidx.at[j]], ssems[b], add=True)
        for b in range(NBUF):
            jn = (t + 1) * NBUF + b

            @pl.when(jn < NCHUNK)
            def _():
                scatter_wait(t * NBUF + b, b)
                pltpu.async_copy(s_hbm.at[ridx.at[jn]], bufs[b], gsems[b])
        return 0
    lax.fori_loop(0, NCHUNK // NBUF, body, 0)

    for b in range(NBUF):
        scatter_wait(NCHUNK - NBUF + b, b)

    plsc.subcore_barrier()
    pltpu.sync_copy(acc.at[pl.ds(s * RPT, RPT)],
                    q_hbm.at[c, pl.ds(s * RPT, RPT)])


_BLK = 2000


def _prep_body(p0_ref, p1_ref, x_ref, disb_ref, s_ref):
    deg = p0_ref[:, 0:1] + p1_ref[:, 0:1]
    dis = jnp.where(deg > 0.0, 1.0 / jnp.sqrt(deg), 0.0)
    dis_b = jnp.broadcast_to(dis, (_BLK, D))
    disb_ref[...] = dis_b
    s_ref[...] = dis_b * x_ref[...]


def _prep_tc(p0, p1, x0):
    return pl.pallas_call(
        _prep_body,
        grid=(N // _BLK,),
        in_specs=[
            pl.BlockSpec((_BLK, 16), lambda i: (i, 0)),
            pl.BlockSpec((_BLK, 16), lambda i: (i, 0)),
            pl.BlockSpec((_BLK, D), lambda i: (i, 0)),
        ],
        out_specs=[
            pl.BlockSpec((_BLK, D), lambda i: (i, 0)),
            pl.BlockSpec((_BLK, D), lambda i: (i, 0)),
        ],
        out_shape=[
            jax.ShapeDtypeStruct((N, D), jnp.float32),
            jax.ShapeDtypeStruct((N, D), jnp.float32),
        ],
    )(p0, p1, x0)


def _combine_body(q0_ref, q1_ref, disb_ref, acc_ref, accn_ref, sn_ref):
    dis_b = disb_ref[...]
    x_next = dis_b * (q0_ref[...] + q1_ref[...])
    accn_ref[...] = acc_ref[...] + x_next
    sn_ref[...] = dis_b * x_next


def _combine_tc(q0, q1, dis_b, acc):
    blk = pl.BlockSpec((_BLK, D), lambda i: (i, 0))
    return pl.pallas_call(
        _combine_body,
        grid=(N // _BLK,),
        in_specs=[blk, blk, blk, blk],
        out_specs=[blk, blk],
        out_shape=[
            jax.ShapeDtypeStruct((N, D), jnp.float32),
            jax.ShapeDtypeStruct((N, D), jnp.float32),
        ],
    )(q0, q1, dis_b, acc)


def _final_body(q0_ref, q1_ref, disb_ref, acc_ref, out_ref):
    x_next = disb_ref[...] * (q0_ref[...] + q1_ref[...])
    out_ref[...] = (acc_ref[...] + x_next) * (1.0 / (NUM_LAYERS + 1))


def _final_tc(q0, q1, dis_b, acc):
    blk = pl.BlockSpec((_BLK, D), lambda i: (i, 0))
    return pl.pallas_call(
        _final_body,
        grid=(N // _BLK,),
        in_specs=[blk, blk, blk, blk],
        out_specs=blk,
        out_shape=jax.ShapeDtypeStruct((N, D), jnp.float32),
    )(q0, q1, dis_b, acc)


def kernel(embedding_weight, edge_index_mp):
    x0 = embedding_weight
    row = edge_index_mp[0].reshape(NW, NCHUNK, K)
    col = edge_index_mp[1].reshape(NW, NCHUNK, K)
    z16 = jnp.zeros((RPT, 16), jnp.float32)
    z = jnp.zeros((RPT, D), jnp.float32)

    degq = _deg_sc(col, z16)
    dis_b, s = _prep_tc(degq[0], degq[1], x0)
    acc = x0
    for l in range(NUM_LAYERS):
        q = _layer_sc(s, row, col, z)
        if l < NUM_LAYERS - 1:
            acc, s = _combine_tc(q[0], q[1], dis_b, acc)
        else:
            out = _final_tc(q[0], q[1], dis_b, acc)
    return out


# speedup vs baseline: 18.9877x; 1.2013x over previous
import functools

import jax
import jax.numpy as jnp
from jax import lax
from jax.experimental import pallas as pl
from jax.experimental.pallas import tpu as pltpu
from jax.experimental.pallas import tpu_sc as plsc

N = 10000
D = 128
E = 320000
NUM_LAYERS = 3

NC = 2
NS = 16
NW = NC * NS
EPT = E // NW
K = 50
NCHUNK = EPT // K
NBUF = 4
RPT = N // NS

_mesh = plsc.VectorSubcoreMesh(
    core_axis_name="c", subcore_axis_name="s", num_cores=NC, num_subcores=NS)

_sc_params = pltpu.CompilerParams(use_tc_tiling_on_sc=False)


@functools.partial(
    pl.kernel,
    out_type=jax.ShapeDtypeStruct((NC, N, 16), jnp.float32),
    mesh=_mesh,
    scratch_types=[
        pltpu.VMEM_SHARED((N, 16), jnp.float32),
        pltpu.VMEM((NCHUNK, K), jnp.int32),
        pltpu.VMEM((K, 16), jnp.float32),
        pltpu.SemaphoreType.DMA,
    ],
    compiler_params=_sc_params,
)
def _deg_sc(col_hbm, z16_hbm, deg_hbm, acc, cidx, ones, sem):
    c = lax.axis_index("c")
    s = lax.axis_index("s")
    tid = c * NS + s

    pltpu.sync_copy(col_hbm.at[tid], cidx)

    ones16 = jnp.full((16,), 1.0, jnp.float32)

    def fill(i, _):
        ones[i, :] = ones16
        return 0
    lax.fori_loop(0, K, fill, 0)

    pltpu.sync_copy(z16_hbm, acc.at[pl.ds(s * RPT, RPT)])
    plsc.subcore_barrier()

    def chunk(j, _):
        pltpu.async_copy(ones, acc.at[cidx.at[j]], sem, add=True)
        return 0
    lax.fori_loop(0, NCHUNK, chunk, 0)

    def drain(j, _):
        pltpu.make_async_copy(ones, acc.at[cidx.at[j]], sem).wait()
        return 0
    lax.fori_loop(0, NCHUNK, drain, 0)

    plsc.subcore_barrier()
    pltpu.sync_copy(acc.at[pl.ds(s * RPT, RPT)],
                    deg_hbm.at[c, pl.ds(s * RPT, RPT)])


@functools.partial(
    pl.kernel,
    out_type=jax.ShapeDtypeStruct((NC, N, D), jnp.float32),
    mesh=_mesh,
    scratch_types=[
        pltpu.VMEM_SHARED((N, D), jnp.float32),
        pltpu.VMEM((NCHUNK, K), jnp.int32),
        pltpu.VMEM((NCHUNK, K), jnp.int32),
        [pltpu.VMEM((K, D), jnp.float32) for _ in range(NBUF)],
        [pltpu.SemaphoreType.DMA for _ in range(NBUF)],
        [pltpu.SemaphoreType.DMA for _ in range(NBUF)],
    ],
    compiler_params=_sc_params,
)
def _layer_sc(s_hbm, row_hbm, col_hbm, z_hbm, q_hbm, acc, ridx, cidx, bufs,
              gsems, ssems):
    c = lax.axis_index("c")
    s = lax.axis_index("s")
    tid = c * NS + s

    pltpu.sync_copy(row_hbm.at[tid], ridx)
    pltpu.sync_copy(col_hbm.at[tid], cidx)

    def gather_wait(j, b):
        pltpu.make_async_copy(s_hbm.at[ridx.at[j]], bufs[b], gsems[b]).wait()

    def scatter_wait(j, b):
        pltpu.make_async_copy(bufs[b], acc.at[cidx.at[j]], ssems[b]).wait()

    for b in range(NBUF):
        pltpu.async_copy(s_hbm.at[ridx.at[b]], bufs[b], gsems[b])

    pltpu.sync_copy(z_hbm, acc.at[pl.ds(s * RPT, RPT)])
    plsc.subcore_barrier()

    def body(t, _):
        for b in range(NBUF):
            j = t * NBUF + b
            gather_wait(j, b)
            pltpu.async_copy(bufs[b], acc.at[cidx.at[j]], ssems[b], add=True)
        for b in range(NBUF):
            jn = (t + 1) * NBUF + b

            @pl.when(jn < NCHUNK)
            def _():
                scatter_wait(t * NBUF + b, b)
                pltpu.async_copy(s_hbm.at[ridx.at[jn]], bufs[b], gsems[b])
        return 0
    lax.fori_loop(0, NCHUNK // NBUF, body, 0)

    for b in range(NBUF):
        scatter_wait(NCHUNK - NBUF + b, b)

    plsc.subcore_barrier()
    pltpu.sync_copy(acc.at[pl.ds(s * RPT, RPT)],
                    q_hbm.at[c, pl.ds(s * RPT, RPT)])


_BLK = 2000


def _prep_body(p0_ref, p1_ref, x_ref, disb_ref, s_ref):
    deg = p0_ref[:, 0:1] + p1_ref[:, 0:1]
    dis = jnp.where(deg > 0.0, 1.0 / jnp.sqrt(deg), 0.0)
    dis_b = jnp.broadcast_to(dis, (_BLK, D))
    disb_ref[...] = dis_b
    s_ref[...] = dis_b * x_ref[...]


def _prep_tc(p0, p1, x0):
    return pl.pallas_call(
        _prep_body,
        grid=(N // _BLK,),
        in_specs=[
            pl.BlockSpec((_BLK, 16), lambda i: (i, 0)),
            pl.BlockSpec((_BLK, 16), lambda i: (i, 0)),
            pl.BlockSpec((_BLK, D), lambda i: (i, 0)),
        ],
        out_specs=[
            pl.BlockSpec((_BLK, D), lambda i: (i, 0)),
            pl.BlockSpec((_BLK, D), lambda i: (i, 0)),
        ],
        out_shape=[
            jax.ShapeDtypeStruct((N, D), jnp.float32),
            jax.ShapeDtypeStruct((N, D), jnp.float32),
        ],
    )(p0, p1, x0)


def _combine_body(q0_ref, q1_ref, disb_ref, acc_ref, accn_ref, sn_ref):
    dis_b = disb_ref[...]
    x_next = dis_b * (q0_ref[...] + q1_ref[...])
    accn_ref[...] = acc_ref[...] + x_next
    sn_ref[...] = dis_b * x_next


def _combine_tc(q0, q1, dis_b, acc):
    blk = pl.BlockSpec((_BLK, D), lambda i: (i, 0))
    return pl.pallas_call(
        _combine_body,
        grid=(N // _BLK,),
        in_specs=[blk, blk, blk, blk],
        out_specs=[blk, blk],
        out_shape=[
            jax.ShapeDtypeStruct((N, D), jnp.float32),
            jax.ShapeDtypeStruct((N, D), jnp.float32),
        ],
    )(q0, q1, dis_b, acc)


def _final_body(q0_ref, q1_ref, disb_ref, acc_ref, out_ref):
    x_next = disb_ref[...] * (q0_ref[...] + q1_ref[...])
    out_ref[...] = (acc_ref[...] + x_next) * (1.0 / (NUM_LAYERS + 1))


def _final_tc(q0, q1, dis_b, acc):
    blk = pl.BlockSpec((_BLK, D), lambda i: (i, 0))
    return pl.pallas_call(
        _final_body,
        grid=(N // _BLK,),
        in_specs=[blk, blk, blk, blk],
        out_specs=blk,
        out_shape=jax.ShapeDtypeStruct((N, D), jnp.float32),
    )(q0, q1, dis_b, acc)


def kernel(embedding_weight, edge_index_mp):
    x0 = embedding_weight
    row = edge_index_mp[0].reshape(NW, NCHUNK, K)
    col = edge_index_mp[1].reshape(NW, NCHUNK, K)
    z16 = jnp.zeros((RPT, 16), jnp.float32)
    z = jnp.zeros((RPT, D), jnp.float32)

    degq = _deg_sc(col, z16)
    dis_b, s = _prep_tc(degq[0], degq[1], x0)
    acc = x0
    for l in range(NUM_LAYERS):
        q = _layer_sc(s, row, col, z)
        if l < NUM_LAYERS - 1:
            acc, s = _combine_tc(q[0], q[1], dis_b, acc)
        else:
            out = _final_tc(q[0], q[1], dis_b, acc)
    return out


# speedup vs baseline: 19.3742x; 1.0204x over previous
import functools

import jax
import jax.numpy as jnp
from jax import lax
from jax.experimental import pallas as pl
from jax.experimental.pallas import tpu as pltpu
from jax.experimental.pallas import tpu_sc as plsc

N = 10000
D = 128
E = 320000
NUM_LAYERS = 3

NC = 2
NS = 16
NW = NC * NS
EPT = E // NW
K = 40
NCHUNK = EPT // K
NBUF = 5
RPT = N // NS

_mesh = plsc.VectorSubcoreMesh(
    core_axis_name="c", subcore_axis_name="s", num_cores=NC, num_subcores=NS)

_sc_params = pltpu.CompilerParams(use_tc_tiling_on_sc=False)


@functools.partial(
    pl.kernel,
    out_type=jax.ShapeDtypeStruct((NC, N, 16), jnp.float32),
    mesh=_mesh,
    scratch_types=[
        pltpu.VMEM_SHARED((N, 16), jnp.float32),
        pltpu.VMEM((NCHUNK, K), jnp.int32),
        pltpu.VMEM((K, 16), jnp.float32),
        pltpu.SemaphoreType.DMA,
    ],
    compiler_params=_sc_params,
)
def _deg_sc(col_hbm, z16_hbm, deg_hbm, acc, cidx, ones, sem):
    c = lax.axis_index("c")
    s = lax.axis_index("s")
    tid = c * NS + s

    pltpu.sync_copy(col_hbm.at[tid], cidx)

    ones16 = jnp.full((16,), 1.0, jnp.float32)

    def fill(i, _):
        ones[i, :] = ones16
        return 0
    lax.fori_loop(0, K, fill, 0)

    pltpu.sync_copy(z16_hbm, acc.at[pl.ds(s * RPT, RPT)])
    plsc.subcore_barrier()

    def chunk(j, _):
        pltpu.async_copy(ones, acc.at[cidx.at[j]], sem, add=True)
        return 0
    lax.fori_loop(0, NCHUNK, chunk, 0)

    def drain(j, _):
        pltpu.make_async_copy(ones, acc.at[cidx.at[j]], sem).wait()
        return 0
    lax.fori_loop(0, NCHUNK, drain, 0)

    plsc.subcore_barrier()
    pltpu.sync_copy(acc.at[pl.ds(s * RPT, RPT)],
                    deg_hbm.at[c, pl.ds(s * RPT, RPT)])


@functools.partial(
    pl.kernel,
    out_type=jax.ShapeDtypeStruct((NC, N, D), jnp.float32),
    mesh=_mesh,
    scratch_types=[
        pltpu.VMEM_SHARED((N, D), jnp.float32),
        pltpu.VMEM((NCHUNK, K), jnp.int32),
        pltpu.VMEM((NCHUNK, K), jnp.int32),
        [pltpu.VMEM((K, D), jnp.float32) for _ in range(NBUF)],
        [pltpu.SemaphoreType.DMA for _ in range(NBUF)],
        [pltpu.SemaphoreType.DMA for _ in range(NBUF)],
    ],
    compiler_params=_sc_params,
)
def _layer_sc(s_hbm, row_hbm, col_hbm, z_hbm, q_hbm, acc, ridx, cidx, bufs,
              gsems, ssems):
    c = lax.axis_index("c")
    s = lax.axis_index("s")
    tid = c * NS + s

    pltpu.sync_copy(row_hbm.at[tid], ridx)
    pltpu.sync_copy(col_hbm.at[tid], cidx)

    def gather_wait(j, b):
        pltpu.make_async_copy(s_hbm.at[ridx.at[j]], bufs[b], gsems[b]).wait()

    def scatter_wait(j, b):
        pltpu.make_async_copy(bufs[b], acc.at[cidx.at[j]], ssems[b]).wait()

    for b in range(NBUF):
        pltpu.async_copy(s_hbm.at[ridx.at[b]], bufs[b], gsems[b])

    pltpu.sync_copy(z_hbm, acc.at[pl.ds(s * RPT, RPT)])
    plsc.subcore_barrier()

    def body(t, _):
        for b in range(NBUF):
            j = t * NBUF + b
            gather_wait(j, b)
            pltpu.async_copy(bufs[b], acc.at[cidx.at[j]], ssems[b], add=True)
        for b in range(NBUF):
            jn = (t + 1) * NBUF + b

            @pl.when(jn < NCHUNK)
            def _():
                scatter_wait(t * NBUF + b, b)
                pltpu.async_copy(s_hbm.at[ridx.at[jn]], bufs[b], gsems[b])
        return 0
    lax.fori_loop(0, NCHUNK // NBUF, body, 0)

    for b in range(NBUF):
        scatter_wait(NCHUNK - NBUF + b, b)

    plsc.subcore_barrier()
    pltpu.sync_copy(acc.at[pl.ds(s * RPT, RPT)],
                    q_hbm.at[c, pl.ds(s * RPT, RPT)])


_BLK = 2000


def _prep_body(p0_ref, p1_ref, x_ref, disb_ref, s_ref):
    deg = p0_ref[:, 0:1] + p1_ref[:, 0:1]
    dis = jnp.where(deg > 0.0, 1.0 / jnp.sqrt(deg), 0.0)
    dis_b = jnp.broadcast_to(dis, (_BLK, D))
    disb_ref[...] = dis_b
    s_ref[...] = dis_b * x_ref[...]


def _prep_tc(p0, p1, x0):
    return pl.pallas_call(
        _prep_body,
        grid=(N // _BLK,),
        in_specs=[
            pl.BlockSpec((_BLK, 16), lambda i: (i, 0)),
            pl.BlockSpec((_BLK, 16), lambda i: (i, 0)),
            pl.BlockSpec((_BLK, D), lambda i: (i, 0)),
        ],
        out_specs=[
            pl.BlockSpec((_BLK, D), lambda i: (i, 0)),
            pl.BlockSpec((_BLK, D), lambda i: (i, 0)),
        ],
        out_shape=[
            jax.ShapeDtypeStruct((N, D), jnp.float32),
            jax.ShapeDtypeStruct((N, D), jnp.float32),
        ],
    )(p0, p1, x0)


def _combine_body(q0_ref, q1_ref, disb_ref, acc_ref, accn_ref, sn_ref):
    dis_b = disb_ref[...]
    x_next = dis_b * (q0_ref[...] + q1_ref[...])
    accn_ref[...] = acc_ref[...] + x_next
    sn_ref[...] = dis_b * x_next


def _combine_tc(q0, q1, dis_b, acc):
    blk = pl.BlockSpec((_BLK, D), lambda i: (i, 0))
    return pl.pallas_call(
        _combine_body,
        grid=(N // _BLK,),
        in_specs=[blk, blk, blk, blk],
        out_specs=[blk, blk],
        out_shape=[
            jax.ShapeDtypeStruct((N, D), jnp.float32),
            jax.ShapeDtypeStruct((N, D), jnp.float32),
        ],
    )(q0, q1, dis_b, acc)


def _final_body(q0_ref, q1_ref, disb_ref, acc_ref, out_ref):
    x_next = disb_ref[...] * (q0_ref[...] + q1_ref[...])
    out_ref[...] = (acc_ref[...] + x_next) * (1.0 / (NUM_LAYERS + 1))


def _final_tc(q0, q1, dis_b, acc):
    blk = pl.BlockSpec((_BLK, D), lambda i: (i, 0))
    return pl.pallas_call(
        _final_body,
        grid=(N // _BLK,),
        in_specs=[blk, blk, blk, blk],
        out_specs=blk,
        out_shape=jax.ShapeDtypeStruct((N, D), jnp.float32),
    )(q0, q1, dis_b, acc)


def kernel(embedding_weight, edge_index_mp):
    x0 = embedding_weight
    row = edge_index_mp[0].reshape(NW, NCHUNK, K)
    col = edge_index_mp[1].reshape(NW, NCHUNK, K)
    z16 = jnp.zeros((RPT, 16), jnp.float32)
    z = jnp.zeros((RPT, D), jnp.float32)

    degq = _deg_sc(col, z16)
    dis_b, s = _prep_tc(degq[0], degq[1], x0)
    acc = x0
    for l in range(NUM_LAYERS):
        q = _layer_sc(s, row, col, z)
        if l < NUM_LAYERS - 1:
            acc, s = _combine_tc(q[0], q[1], dis_b, acc)
        else:
            out = _final_tc(q[0], q[1], dis_b, acc)
    return out
